# Initial kernel scaffold; baseline (speedup 1.0000x reference)
import jax
import jax.numpy as jnp
from jax.experimental import pallas as pl


def kernel(x, edge_index, edge_attr, u, batch, W, b):
    raise NotImplementedError("write your pallas kernel here")


# speedup vs baseline: 13.1171x; 13.1171x over previous
import functools

import jax
import jax.numpy as jnp
from jax import lax
from jax.experimental import pallas as pl
from jax.experimental.pallas import tpu as pltpu
from jax.experimental.pallas import tpu_sc as plsc

N = 10000
E = 320000
D_EDGE = 16
CHUNK = 128
NCHUNKS = E // CHUNK
NW = 32
TILE_CHUNKS = 80
LAST_CHUNKS = NCHUNKS - TILE_CHUNKS * (NW - 1)
IDX_PAD_ROWS = TILE_CHUNKS * NW
KBUF = 5
OUT_SLICE = 624


def _sc_segment_sum(idx2d, attr3d, zeros):
    mesh = plsc.VectorSubcoreMesh(core_axis_name="c", subcore_axis_name="s")

    @functools.partial(
        pl.kernel,
        mesh=mesh,
        compiler_params=pltpu.CompilerParams(use_tc_tiling_on_sc=False),
        out_type=jax.ShapeDtypeStruct((2 * N, D_EDGE), jnp.float32),
        scratch_types=[
            pltpu.VMEM((KBUF, 1, CHUNK), jnp.int32),
            pltpu.VMEM((KBUF * CHUNK, D_EDGE), jnp.float32),
            pltpu.VMEM_SHARED((N, D_EDGE), jnp.float32),
        ],
    )
    def body(idx_hbm, attr_hbm, zeros_hbm, out_hbm, idx_v, rows_v, acc_sh):
        cid = lax.axis_index("c")
        sid = lax.axis_index("s")
        wid = cid * 16 + sid
        start = wid * TILE_CHUNKS
        niter = jnp.where(wid == NW - 1, LAST_CHUNKS // KBUF,
                          TILE_CHUNKS // KBUF)

        @pl.when(sid == 0)
        def _():
            pltpu.sync_copy(zeros_hbm, acc_sh)

        plsc.subcore_barrier()

        def outer(g, carry):
            pltpu.sync_copy(idx_hbm.at[pl.ds(start + g * KBUF, KBUF)], idx_v)
            pltpu.sync_copy(
                attr_hbm.at[pl.ds((start + g * KBUF) * CHUNK, KBUF * CHUNK)],
                rows_v)
            for j in range(KBUF):
                pltpu.sync_copy(rows_v.at[pl.ds(j * CHUNK, CHUNK)],
                                acc_sh.at[idx_v.at[j, 0]],
                                add=True)
            return carry

        lax.fori_loop(0, niter, outer, 0)

        plsc.subcore_barrier()

        row0 = cid * N + sid * OUT_SLICE
        pltpu.sync_copy(acc_sh.at[pl.ds(sid * OUT_SLICE, OUT_SLICE)],
                        out_hbm.at[pl.ds(row0, OUT_SLICE)])

        @pl.when(sid == 15)
        def _():
            tail = N - 16 * OUT_SLICE
            pltpu.sync_copy(acc_sh.at[pl.ds(16 * OUT_SLICE, tail)],
                            out_hbm.at[pl.ds(cid * N + 16 * OUT_SLICE, tail)])

    return body(idx2d, attr3d, zeros)


def _mlp_body(x_ref, p_ref, u_ref, w1_ref, w2_ref, w3_ref, b_ref, o_ref):
    seg = p_ref[0] + p_ref[1]
    acc = jnp.dot(x_ref[...], w1_ref[...], preferred_element_type=jnp.float32)
    acc += jnp.dot(seg, w2_ref[...], preferred_element_type=jnp.float32)
    acc += jnp.dot(u_ref[...], w3_ref[...], preferred_element_type=jnp.float32)
    acc += b_ref[...]
    o_ref[...] = jnp.maximum(acc, 0.0)


def _tc_mlp(x, partial, u, w1t, w2t, w3t, b2d):
    blk = 1000
    grid = (N // blk,)
    return pl.pallas_call(
        _mlp_body,
        grid=grid,
        in_specs=[
            pl.BlockSpec((blk, 128), lambda i: (i, 0)),
            pl.BlockSpec((2, blk, D_EDGE), lambda i: (0, i, 0)),
            pl.BlockSpec((blk, 64), lambda i: (i, 0)),
            pl.BlockSpec((128, 128), lambda i: (0, 0)),
            pl.BlockSpec((D_EDGE, 128), lambda i: (0, 0)),
            pl.BlockSpec((64, 128), lambda i: (0, 0)),
            pl.BlockSpec((1, 128), lambda i: (0, 0)),
        ],
        out_specs=pl.BlockSpec((blk, 128), lambda i: (i, 0)),
        out_shape=jax.ShapeDtypeStruct((N, 128), jnp.float32),
    )(x, partial, u, w1t, w2t, w3t, b2d)


def kernel(x, edge_index, edge_attr, u, batch, W, b):
    idx2d = edge_index[0].reshape(NCHUNKS, CHUNK).astype(jnp.int32)
    idx2d = jnp.pad(idx2d, ((0, IDX_PAD_ROWS - NCHUNKS), (0, 0)))
    idx2d = idx2d.reshape(IDX_PAD_ROWS, 1, CHUNK)
    attr3d = edge_attr
    zeros = jnp.zeros((N, D_EDGE), jnp.float32)
    partial = _sc_segment_sum(idx2d, attr3d, zeros).reshape(2, N, D_EDGE)
    w1t = W[:, :128].T
    w2t = W[:, 128:144].T
    w3t = W[:, 144:].T
    b2d = b.reshape(1, 128)
    return _tc_mlp(x, partial, u, w1t, w2t, w3t, b2d)


# speedup vs baseline: 14.5761x; 1.1112x over previous
import functools

import jax
import jax.numpy as jnp
from jax import lax
from jax.experimental import pallas as pl
from jax.experimental.pallas import tpu as pltpu
from jax.experimental.pallas import tpu_sc as plsc

N = 10000
E = 320000
D_EDGE = 16
CHUNK = 128
NCHUNKS = E // CHUNK
NW = 32
TILE_CHUNKS = 80
LAST_CHUNKS = NCHUNKS - TILE_CHUNKS * (NW - 1)
IDX_PAD_ROWS = TILE_CHUNKS * NW
KBUF = 10
OUT_SLICE = 624


def _sc_segment_sum(idx2d, attr3d, zeros):
    mesh = plsc.VectorSubcoreMesh(core_axis_name="c", subcore_axis_name="s")

    @functools.partial(
        pl.kernel,
        mesh=mesh,
        compiler_params=pltpu.CompilerParams(use_tc_tiling_on_sc=False),
        out_type=jax.ShapeDtypeStruct((2 * N, D_EDGE), jnp.float32),
        scratch_types=[
            pltpu.VMEM((KBUF, 1, CHUNK), jnp.int32),
            pltpu.VMEM((KBUF, 1, CHUNK), jnp.int32),
            pltpu.VMEM((KBUF * CHUNK, D_EDGE), jnp.float32),
            pltpu.VMEM((KBUF * CHUNK, D_EDGE), jnp.float32),
            pltpu.VMEM_SHARED((N, D_EDGE), jnp.float32),
            pltpu.SemaphoreType.DMA,
            pltpu.SemaphoreType.DMA,
            pltpu.SemaphoreType.DMA,
            pltpu.SemaphoreType.DMA,
        ],
    )
    def body(idx_hbm, attr_hbm, zeros_hbm, out_hbm, idx_v0, idx_v1,
             rows_v0, rows_v1, acc_sh, sg0, sg1, ss0, ss1):
        cid = lax.axis_index("c")
        sid = lax.axis_index("s")
        wid = cid * 16 + sid
        start = wid * TILE_CHUNKS
        niter = jnp.where(wid == NW - 1, LAST_CHUNKS // KBUF,
                          TILE_CHUNKS // KBUF)

        def idx_src(g):
            return idx_hbm.at[pl.ds(start + g * KBUF, KBUF)]

        def attr_src(g):
            return attr_hbm.at[pl.ds((start + g * KBUF) * CHUNK,
                                     KBUF * CHUNK)]

        def start_gather(g, idxbuf, rowbuf, sem):
            pltpu.make_async_copy(idx_src(g), idxbuf, sem).start()
            pltpu.make_async_copy(attr_src(g), rowbuf, sem).start()

        def wait_gather(g, idxbuf, rowbuf, sem):
            pltpu.make_async_copy(idx_src(g), idxbuf, sem).wait()
            pltpu.make_async_copy(attr_src(g), rowbuf, sem).wait()

        def fire_scatters(idxbuf, rowbuf, sem):
            return [pltpu.async_copy(rowbuf.at[pl.ds(j * CHUNK, CHUNK)],
                                     acc_sh.at[idxbuf.at[j, 0]],
                                     sem, add=True)
                    for j in range(KBUF)]

        start_gather(0, idx_v0, rows_v0, sg0)
        start_gather(1, idx_v1, rows_v1, sg1)

        @pl.when(sid == 0)
        def _():
            pltpu.sync_copy(zeros_hbm, acc_sh)

        plsc.subcore_barrier()

        def outer(h, carry):
            g0 = 2 * h
            wait_gather(g0, idx_v0, rows_v0, sg0)
            sc0 = fire_scatters(idx_v0, rows_v0, ss0)
            wait_gather(g0 + 1, idx_v1, rows_v1, sg1)
            sc1 = fire_scatters(idx_v1, rows_v1, ss1)
            for c in sc0:
                c.wait()

            @pl.when(g0 + 2 < niter)
            def _():
                start_gather(g0 + 2, idx_v0, rows_v0, sg0)

            for c in sc1:
                c.wait()

            @pl.when(g0 + 3 < niter)
            def _():
                start_gather(g0 + 3, idx_v1, rows_v1, sg1)

            return carry

        lax.fori_loop(0, niter // 2, outer, 0)

        plsc.subcore_barrier()

        row0 = cid * N + sid * OUT_SLICE
        pltpu.sync_copy(acc_sh.at[pl.ds(sid * OUT_SLICE, OUT_SLICE)],
                        out_hbm.at[pl.ds(row0, OUT_SLICE)])

        @pl.when(sid == 15)
        def _():
            tail = N - 16 * OUT_SLICE
            pltpu.sync_copy(acc_sh.at[pl.ds(16 * OUT_SLICE, tail)],
                            out_hbm.at[pl.ds(cid * N + 16 * OUT_SLICE, tail)])

    return body(idx2d, attr3d, zeros)


def _mlp_body(x_ref, p_ref, u_ref, w1_ref, w2_ref, w3_ref, b_ref, o_ref):
    seg = p_ref[0] + p_ref[1]
    acc = jnp.dot(x_ref[...], w1_ref[...], preferred_element_type=jnp.float32)
    acc += jnp.dot(seg, w2_ref[...], preferred_element_type=jnp.float32)
    acc += jnp.dot(u_ref[...], w3_ref[...], preferred_element_type=jnp.float32)
    acc += b_ref[...]
    o_ref[...] = jnp.maximum(acc, 0.0)


def _tc_mlp(x, partial, u, w1t, w2t, w3t, b2d):
    blk = 1000
    grid = (N // blk,)
    return pl.pallas_call(
        _mlp_body,
        grid=grid,
        in_specs=[
            pl.BlockSpec((blk, 128), lambda i: (i, 0)),
            pl.BlockSpec((2, blk, D_EDGE), lambda i: (0, i, 0)),
            pl.BlockSpec((blk, 64), lambda i: (i, 0)),
            pl.BlockSpec((128, 128), lambda i: (0, 0)),
            pl.BlockSpec((D_EDGE, 128), lambda i: (0, 0)),
            pl.BlockSpec((64, 128), lambda i: (0, 0)),
            pl.BlockSpec((1, 128), lambda i: (0, 0)),
        ],
        out_specs=pl.BlockSpec((blk, 128), lambda i: (i, 0)),
        out_shape=jax.ShapeDtypeStruct((N, 128), jnp.float32),
    )(x, partial, u, w1t, w2t, w3t, b2d)


def kernel(x, edge_index, edge_attr, u, batch, W, b):
    idx2d = edge_index[0].reshape(NCHUNKS, CHUNK).astype(jnp.int32)
    idx2d = jnp.pad(idx2d, ((0, IDX_PAD_ROWS - NCHUNKS), (0, 0)))
    idx2d = idx2d.reshape(IDX_PAD_ROWS, 1, CHUNK)
    attr3d = edge_attr
    zeros = jnp.zeros((N, D_EDGE), jnp.float32)
    partial = _sc_segment_sum(idx2d, attr3d, zeros).reshape(2, N, D_EDGE)
    w1t = W[:, :128].T
    w2t = W[:, 128:144].T
    w3t = W[:, 144:].T
    b2d = b.reshape(1, 128)
    return _tc_mlp(x, partial, u, w1t, w2t, w3t, b2d)


# speedup vs baseline: 14.6739x; 1.0067x over previous
import functools

import jax
import jax.numpy as jnp
from jax import lax
from jax.experimental import pallas as pl
from jax.experimental.pallas import tpu as pltpu
from jax.experimental.pallas import tpu_sc as plsc

N = 10000
E = 320000
D_EDGE = 16
CHUNK = 128
NCHUNKS = E // CHUNK
NW = 32
TILE_CHUNKS = 80
LAST_CHUNKS = NCHUNKS - TILE_CHUNKS * (NW - 1)
IDX_PAD_ROWS = TILE_CHUNKS * NW
KBUF = 10
OUT_SLICE = 624


def _sc_segment_sum(idx2d, attr3d, zeros):
    mesh = plsc.VectorSubcoreMesh(core_axis_name="c", subcore_axis_name="s")

    @functools.partial(
        pl.kernel,
        mesh=mesh,
        compiler_params=pltpu.CompilerParams(use_tc_tiling_on_sc=False),
        out_type=jax.ShapeDtypeStruct((2 * N, D_EDGE), jnp.float32),
        scratch_types=[
            pltpu.VMEM((KBUF, 1, CHUNK), jnp.int32),
            pltpu.VMEM((KBUF, 1, CHUNK), jnp.int32),
            pltpu.VMEM((KBUF * CHUNK, D_EDGE), jnp.float32),
            pltpu.VMEM((KBUF * CHUNK, D_EDGE), jnp.float32),
            pltpu.VMEM_SHARED((N, D_EDGE), jnp.float32),
            pltpu.SemaphoreType.DMA,
            pltpu.SemaphoreType.DMA,
            pltpu.SemaphoreType.DMA,
            pltpu.SemaphoreType.DMA,
        ],
    )
    def body(idx_hbm, attr_hbm, zeros_hbm, out_hbm, idx_v0, idx_v1,
             rows_v0, rows_v1, acc_sh, sg0, sg1, ss0, ss1):
        cid = lax.axis_index("c")
        sid = lax.axis_index("s")
        wid = cid * 16 + sid
        start = wid * TILE_CHUNKS
        niter = jnp.where(wid == NW - 1, LAST_CHUNKS // KBUF,
                          TILE_CHUNKS // KBUF)

        def idx_src(g):
            return idx_hbm.at[pl.ds(start + g * KBUF, KBUF)]

        def attr_src(g):
            return attr_hbm.at[pl.ds((start + g * KBUF) * CHUNK,
                                     KBUF * CHUNK)]

        def start_gather(g, idxbuf, rowbuf, sem):
            pltpu.make_async_copy(idx_src(g), idxbuf, sem).start()
            pltpu.make_async_copy(attr_src(g), rowbuf, sem).start()

        def wait_gather(g, idxbuf, rowbuf, sem):
            pltpu.make_async_copy(idx_src(g), idxbuf, sem).wait()
            pltpu.make_async_copy(attr_src(g), rowbuf, sem).wait()

        def fire_scatters(idxbuf, rowbuf, sem):
            return [pltpu.async_copy(rowbuf.at[pl.ds(j * CHUNK, CHUNK)],
                                     acc_sh.at[idxbuf.at[j, 0]],
                                     sem, add=True)
                    for j in range(KBUF)]

        start_gather(0, idx_v0, rows_v0, sg0)
        start_gather(1, idx_v1, rows_v1, sg1)

        @pl.when(sid == 0)
        def _():
            pltpu.sync_copy(zeros_hbm, acc_sh)

        plsc.subcore_barrier()

        def outer(h, carry):
            g0 = 2 * h
            wait_gather(g0, idx_v0, rows_v0, sg0)
            sc0 = fire_scatters(idx_v0, rows_v0, ss0)
            wait_gather(g0 + 1, idx_v1, rows_v1, sg1)
            sc1 = fire_scatters(idx_v1, rows_v1, ss1)
            for c in sc0:
                c.wait()

            @pl.when(g0 + 2 < niter)
            def _():
                start_gather(g0 + 2, idx_v0, rows_v0, sg0)

            for c in sc1:
                c.wait()

            @pl.when(g0 + 3 < niter)
            def _():
                start_gather(g0 + 3, idx_v1, rows_v1, sg1)

            return carry

        lax.fori_loop(0, niter // 2, outer, 0)

        plsc.subcore_barrier()

        row0 = cid * N + sid * OUT_SLICE
        pltpu.sync_copy(acc_sh.at[pl.ds(sid * OUT_SLICE, OUT_SLICE)],
                        out_hbm.at[pl.ds(row0, OUT_SLICE)])

        @pl.when(sid == 15)
        def _():
            tail = N - 16 * OUT_SLICE
            pltpu.sync_copy(acc_sh.at[pl.ds(16 * OUT_SLICE, tail)],
                            out_hbm.at[pl.ds(cid * N + 16 * OUT_SLICE, tail)])

    return body(idx2d, attr3d, zeros)


def _dense_body(x_ref, u_ref, w1_ref, w3_ref, b_ref, o_ref):
    acc = jnp.dot(x_ref[...], w1_ref[...], preferred_element_type=jnp.float32)
    acc += jnp.dot(u_ref[...], w3_ref[...], preferred_element_type=jnp.float32)
    o_ref[...] = acc + b_ref[...]


def _tc_dense(x, u, w1t, w3t, b2d):
    blk = 1000
    return pl.pallas_call(
        _dense_body,
        grid=(N // blk,),
        in_specs=[
            pl.BlockSpec((blk, 128), lambda i: (i, 0)),
            pl.BlockSpec((blk, 64), lambda i: (i, 0)),
            pl.BlockSpec((128, 128), lambda i: (0, 0)),
            pl.BlockSpec((64, 128), lambda i: (0, 0)),
            pl.BlockSpec((1, 128), lambda i: (0, 0)),
        ],
        out_specs=pl.BlockSpec((blk, 128), lambda i: (i, 0)),
        out_shape=jax.ShapeDtypeStruct((N, 128), jnp.float32),
    )(x, u, w1t, w3t, b2d)


def _fini_body(acc_ref, p_ref, w2_ref, o_ref):
    seg = p_ref[0] + p_ref[1]
    acc = acc_ref[...] + jnp.dot(seg, w2_ref[...],
                                 preferred_element_type=jnp.float32)
    o_ref[...] = jnp.maximum(acc, 0.0)


def _tc_finish(acc, partial, w2t):
    blk = 1000
    return pl.pallas_call(
        _fini_body,
        grid=(N // blk,),
        in_specs=[
            pl.BlockSpec((blk, 128), lambda i: (i, 0)),
            pl.BlockSpec((2, blk, D_EDGE), lambda i: (0, i, 0)),
            pl.BlockSpec((D_EDGE, 128), lambda i: (0, 0)),
        ],
        out_specs=pl.BlockSpec((blk, 128), lambda i: (i, 0)),
        out_shape=jax.ShapeDtypeStruct((N, 128), jnp.float32),
    )(acc, partial, w2t)


def kernel(x, edge_index, edge_attr, u, batch, W, b):
    idx2d = edge_index[0].reshape(NCHUNKS, CHUNK).astype(jnp.int32)
    idx2d = jnp.pad(idx2d, ((0, IDX_PAD_ROWS - NCHUNKS), (0, 0)))
    idx2d = idx2d.reshape(IDX_PAD_ROWS, 1, CHUNK)
    attr3d = edge_attr
    zeros = jnp.zeros((N, D_EDGE), jnp.float32)
    partial = _sc_segment_sum(idx2d, attr3d, zeros).reshape(2, N, D_EDGE)
    w1t = W[:, :128].T
    w2t = W[:, 128:144].T
    w3t = W[:, 144:].T
    b2d = b.reshape(1, 128)
    acc = _tc_dense(x, u, w1t, w3t, b2d)
    return _tc_finish(acc, partial, w2t)


# speedup vs baseline: 14.6856x; 1.0008x over previous
import functools

import jax
import jax.numpy as jnp
from jax import lax
from jax.experimental import pallas as pl
from jax.experimental.pallas import tpu as pltpu
from jax.experimental.pallas import tpu_sc as plsc

N = 10000
E = 320000
D_EDGE = 16
CHUNK = 128
NCHUNKS = E // CHUNK
NW = 32
TILE_CHUNKS = 80
LAST_CHUNKS = NCHUNKS - TILE_CHUNKS * (NW - 1)
IDX_PAD_ROWS = TILE_CHUNKS * NW
KBUF = 10
OUT_SLICE = 624


def _sc_segment_sum(idx2d, attr3d, zeros):
    mesh = plsc.VectorSubcoreMesh(core_axis_name="c", subcore_axis_name="s")

    @functools.partial(
        pl.kernel,
        mesh=mesh,
        compiler_params=pltpu.CompilerParams(use_tc_tiling_on_sc=False),
        out_type=jax.ShapeDtypeStruct((2 * N, D_EDGE), jnp.float32),
        scratch_types=[
            pltpu.VMEM((KBUF, 1, CHUNK), jnp.int32),
            pltpu.VMEM((KBUF, 1, CHUNK), jnp.int32),
            pltpu.VMEM((KBUF * CHUNK, D_EDGE), jnp.float32),
            pltpu.VMEM((KBUF * CHUNK, D_EDGE), jnp.float32),
            pltpu.VMEM_SHARED((N, D_EDGE), jnp.float32),
            pltpu.SemaphoreType.DMA,
            pltpu.SemaphoreType.DMA,
            pltpu.SemaphoreType.DMA,
            pltpu.SemaphoreType.DMA,
        ],
    )
    def body(idx_hbm, attr_hbm, zeros_hbm, out_hbm, idx_v0, idx_v1,
             rows_v0, rows_v1, acc_sh, sg0, sg1, ss0, ss1):
        cid = lax.axis_index("c")
        sid = lax.axis_index("s")
        wid = cid * 16 + sid
        start = wid * TILE_CHUNKS
        niter = jnp.where(wid == NW - 1, LAST_CHUNKS // KBUF,
                          TILE_CHUNKS // KBUF)

        def idx_src(g):
            return idx_hbm.at[pl.ds(start + g * KBUF, KBUF)]

        def attr_src(g):
            return attr_hbm.at[pl.ds((start + g * KBUF) * CHUNK,
                                     KBUF * CHUNK)]

        def start_gather(g, idxbuf, rowbuf, sem):
            pltpu.make_async_copy(idx_src(g), idxbuf, sem).start()
            pltpu.make_async_copy(attr_src(g), rowbuf, sem).start()

        def wait_gather(g, idxbuf, rowbuf, sem):
            pltpu.make_async_copy(idx_src(g), idxbuf, sem).wait()
            pltpu.make_async_copy(attr_src(g), rowbuf, sem).wait()

        def fire_scatters(idxbuf, rowbuf, sem):
            return [pltpu.async_copy(rowbuf.at[pl.ds(j * CHUNK, CHUNK)],
                                     acc_sh.at[idxbuf.at[j, 0]],
                                     sem, add=True)
                    for j in range(KBUF)]

        start_gather(0, idx_v0, rows_v0, sg0)
        start_gather(1, idx_v1, rows_v1, sg1)

        @pl.when(sid == 0)
        def _():
            pltpu.sync_copy(zeros_hbm, acc_sh)

        plsc.subcore_barrier()

        def outer(h, carry):
            g0 = 2 * h
            wait_gather(g0, idx_v0, rows_v0, sg0)
            sc0 = fire_scatters(idx_v0, rows_v0, ss0)
            wait_gather(g0 + 1, idx_v1, rows_v1, sg1)
            sc1 = fire_scatters(idx_v1, rows_v1, ss1)
            for c in sc0:
                c.wait()

            @pl.when(g0 + 2 < niter)
            def _():
                start_gather(g0 + 2, idx_v0, rows_v0, sg0)

            for c in sc1:
                c.wait()

            @pl.when(g0 + 3 < niter)
            def _():
                start_gather(g0 + 3, idx_v1, rows_v1, sg1)

            return carry

        lax.fori_loop(0, niter // 2, outer, 0)

        plsc.subcore_barrier()

        row0 = cid * N + sid * OUT_SLICE
        pltpu.sync_copy(acc_sh.at[pl.ds(sid * OUT_SLICE, OUT_SLICE)],
                        out_hbm.at[pl.ds(row0, OUT_SLICE)])

        @pl.when(sid == 15)
        def _():
            tail = N - 16 * OUT_SLICE
            pltpu.sync_copy(acc_sh.at[pl.ds(16 * OUT_SLICE, tail)],
                            out_hbm.at[pl.ds(cid * N + 16 * OUT_SLICE, tail)])

    return body(idx2d, attr3d, zeros)


def _dense_body(x_ref, u_ref, w_ref, b_ref, o_ref):
    dn = (((1,), (1,)), ((), ()))
    acc = lax.dot_general(x_ref[...], w_ref[:, :128], dn,
                          preferred_element_type=jnp.float32)
    acc += lax.dot_general(u_ref[...], w_ref[:, 144:], dn,
                           preferred_element_type=jnp.float32)
    o_ref[...] = acc + b_ref[...]


def _tc_dense(x, u, W, b2d):
    blk = 1000
    return pl.pallas_call(
        _dense_body,
        grid=(N // blk,),
        in_specs=[
            pl.BlockSpec((blk, 128), lambda i: (i, 0)),
            pl.BlockSpec((blk, 64), lambda i: (i, 0)),
            pl.BlockSpec((128, 208), lambda i: (0, 0)),
            pl.BlockSpec((1, 128), lambda i: (0, 0)),
        ],
        out_specs=pl.BlockSpec((blk, 128), lambda i: (i, 0)),
        out_shape=jax.ShapeDtypeStruct((N, 128), jnp.float32),
    )(x, u, W, b2d)


def _fini_body(acc_ref, p_ref, w_ref, o_ref):
    seg = p_ref[0] + p_ref[1]
    dn = (((1,), (1,)), ((), ()))
    acc = acc_ref[...] + lax.dot_general(seg, w_ref[:, 128:144], dn,
                                         preferred_element_type=jnp.float32)
    o_ref[...] = jnp.maximum(acc, 0.0)


def _tc_finish(acc, partial, W):
    blk = 1000
    return pl.pallas_call(
        _fini_body,
        grid=(N // blk,),
        in_specs=[
            pl.BlockSpec((blk, 128), lambda i: (i, 0)),
            pl.BlockSpec((2, blk, D_EDGE), lambda i: (0, i, 0)),
            pl.BlockSpec((128, 208), lambda i: (0, 0)),
        ],
        out_specs=pl.BlockSpec((blk, 128), lambda i: (i, 0)),
        out_shape=jax.ShapeDtypeStruct((N, 128), jnp.float32),
    )(acc, partial, W)


def kernel(x, edge_index, edge_attr, u, batch, W, b):
    idx3d = edge_index[0].reshape(NCHUNKS, 1, CHUNK)
    zeros = jnp.zeros((N, D_EDGE), jnp.float32)
    partial = _sc_segment_sum(idx3d, edge_attr, zeros).reshape(2, N, D_EDGE)
    b2d = b.reshape(1, 128)
    acc = _tc_dense(x, u, W, b2d)
    return _tc_finish(acc, partial, W)
